# Initial kernel scaffold; baseline (speedup 1.0000x reference)
import jax
import jax.numpy as jnp
from jax.experimental import pallas as pl


def kernel(feat, goal_feat, info_feat, adj, params):
    raise NotImplementedError("write your pallas kernel here")


# speedup vs baseline: 27.5516x; 27.5516x over previous
import functools

import jax
import jax.numpy as jnp
from jax.experimental import pallas as pl
from jax.experimental.pallas import tpu as pltpu

N = 10000
H = 256
VIS = 128
RB = 400
GRID_R = N // RB
CB = 2000
GRID_C = N // CB

_f32 = jnp.float32


def _bspec(shape, imap):
    return pl.BlockSpec(shape, imap)


def _mm(xs, ws, bias, affine=None, relu=True, stats=True):
    nx = len(xs)
    M = ws[0].shape[1]

    def body(*refs):
        xrefs = refs[:nx]
        wrefs = refs[nx:2 * nx]
        idx = 2 * nx
        bref = refs[idx]; idx += 1
        if affine is not None:
            saref = refs[idx]; caref = refs[idx + 1]; idx += 2
        oref = refs[idx]; idx += 1
        stref = refs[idx] if stats else None

        x0 = xrefs[0][...]
        if affine is not None:
            x0 = x0 * saref[...] + caref[...]
        acc = jnp.dot(x0, wrefs[0][...], preferred_element_type=_f32)
        for k in range(1, nx):
            acc = acc + jnp.dot(xrefs[k][...], wrefs[k][...],
                                preferred_element_type=_f32)
        acc = acc + bref[...]
        if relu:
            acc = jnp.maximum(acc, 0.0)
        oref[...] = acc
        if stats:
            i = pl.program_id(0)
            upd = jnp.concatenate(
                [jnp.sum(acc, 0)[None], jnp.sum(acc * acc, 0)[None],
                 jnp.zeros((6, M), _f32)], 0)

            @pl.when(i == 0)
            def _():
                stref[...] = upd

            @pl.when(i > 0)
            def _():
                stref[...] = stref[...] + upd

    in_specs = [_bspec((RB, x.shape[1]), lambda i: (i, 0)) for x in xs]
    in_specs += [_bspec(w.shape, lambda i: (0, 0)) for w in ws]
    in_specs += [_bspec((1, M), lambda i: (0, 0))]
    inputs = list(xs) + list(ws) + [bias.reshape(1, M)]
    if affine is not None:
        K0 = xs[0].shape[1]
        in_specs += [_bspec((1, K0), lambda i: (0, 0))] * 2
        inputs += [affine[0].reshape(1, K0), affine[1].reshape(1, K0)]
    out_shape = [jax.ShapeDtypeStruct((N, M), _f32)]
    out_specs = [_bspec((RB, M), lambda i: (i, 0))]
    if stats:
        out_shape.append(jax.ShapeDtypeStruct((8, M), _f32))
        out_specs.append(_bspec((8, M), lambda i: (0, 0)))
    res = pl.pallas_call(
        body, grid=(GRID_R,), in_specs=in_specs, out_specs=out_specs,
        out_shape=out_shape,
        compiler_params=pltpu.CompilerParams(
            dimension_semantics=("arbitrary",)),
    )(*inputs)
    return res if stats else res[0]


def _bn_aff(sums, g, be):
    m = sums[0] / N
    v = sums[1] / N - m * m
    s = g * jax.lax.rsqrt(v + 1e-5)
    return s, be - m * s


def _hst(fx, W, a2):
    def body(fxref, wref, aref, href, stref):
        h = jnp.dot(fxref[...], wref[...], preferred_element_type=_f32)
        href[...] = h
        stref[...] = jnp.dot(h, aref[...], preferred_element_type=_f32)

    return pl.pallas_call(
        body, grid=(GRID_R,),
        in_specs=[_bspec((RB, H), lambda i: (i, 0)),
                  _bspec((H, H), lambda i: (0, 0)),
                  _bspec((H, 2), lambda i: (0, 0))],
        out_specs=[_bspec((RB, H), lambda i: (i, 0)),
                   _bspec((RB, 2), lambda i: (i, 0))],
        out_shape=[jax.ShapeDtypeStruct((N, H), _f32),
                   jax.ShapeDtypeStruct((N, 2), _f32)],
        compiler_params=pltpu.CompilerParams(
            dimension_semantics=("arbitrary",)),
    )(fx, W, a2)


def _gat_dense(adj, st, h):
    GB = 200

    def body(adjref, strref, stcref, href, hpref, rsref):
        s = strref[:, 0]
        t = stcref[:, 1]
        p1 = jnp.exp(-s)[:, None] * jnp.exp(-t)[None, :]
        p2 = jnp.exp(-0.1 * s)[:, None] * jnp.exp(-0.1 * t)[None, :]
        E = adjref[...] * jnp.minimum(p1, p2)
        hpref[...] = jnp.dot(E, href[...], preferred_element_type=_f32)
        rsref[...] = jnp.sum(E, axis=1, keepdims=True)

    return pl.pallas_call(
        body, grid=(N // GB,),
        in_specs=[_bspec((GB, N), lambda i: (i, 0)),
                  _bspec((GB, 2), lambda i: (i, 0)),
                  _bspec((N, 2), lambda i: (0, 0)),
                  _bspec((N, H), lambda i: (0, 0))],
        out_specs=[_bspec((GB, H), lambda i: (i, 0)),
                   _bspec((GB, 1), lambda i: (i, 0))],
        out_shape=[jax.ShapeDtypeStruct((N, H), _f32),
                   jax.ShapeDtypeStruct((N, 1), _f32)],
        compiler_params=pltpu.CompilerParams(
            dimension_semantics=("arbitrary",)),
    )(adj, st, st, h)


def _post_agg(hp, rs):
    def body(hpref, rsref, rref, stref):
        i = pl.program_id(0)
        r = jnp.maximum(hpref[...], 0.0) / (rsref[...] + 1e-5)
        rref[...] = r
        upd = jnp.concatenate(
            [jnp.sum(r, 0)[None], jnp.sum(r * r, 0)[None],
             jnp.zeros((6, H), _f32)], 0)

        @pl.when(i == 0)
        def _():
            stref[...] = upd

        @pl.when(i > 0)
        def _():
            stref[...] = stref[...] + upd

    return pl.pallas_call(
        body, grid=(GRID_R,),
        in_specs=[_bspec((RB, H), lambda i: (i, 0)),
                  _bspec((RB, 1), lambda i: (i, 0))],
        out_specs=[_bspec((RB, H), lambda i: (i, 0)),
                   _bspec((8, H), lambda i: (0, 0))],
        out_shape=[jax.ShapeDtypeStruct((N, H), _f32),
                   jax.ShapeDtypeStruct((8, H), _f32)],
        compiler_params=pltpu.CompilerParams(
            dimension_semantics=("arbitrary",)),
    )(hp, rs)


def _residual(r, fx, sbn, cbn):
    def body(rref, fxref, sref, cref, oref):
        oref[...] = rref[...] * sref[...] + cref[...] + fxref[...]

    return pl.pallas_call(
        body, grid=(GRID_R,),
        in_specs=[_bspec((RB, H), lambda i: (i, 0)),
                  _bspec((RB, H), lambda i: (i, 0)),
                  _bspec((1, H), lambda i: (0, 0)),
                  _bspec((1, H), lambda i: (0, 0))],
        out_specs=_bspec((RB, H), lambda i: (i, 0)),
        out_shape=jax.ShapeDtypeStruct((N, H), _f32),
        compiler_params=pltpu.CompilerParams(
            dimension_semantics=("arbitrary",)),
    )(r, fx, sbn.reshape(1, H), cbn.reshape(1, H))


def _vl_head(o0, o1, o2, goal, info, v):
    w0 = v['w0']
    w0s = [w0[:H], w0[H:2 * H], w0[2 * H:3 * H],
           w0[3 * H:3 * H + VIS], w0[3 * H + VIS:]]

    def body(o0r, o1r, o2r, gr, ir, w0a, w0b, w0c, w0d, w0e,
             b0r, w1r, b1r, w2r, b2r, w3r, b3r, outr):
        x = jnp.dot(o0r[...], w0a[...], preferred_element_type=_f32)
        x += jnp.dot(o1r[...], w0b[...], preferred_element_type=_f32)
        x += jnp.dot(o2r[...], w0c[...], preferred_element_type=_f32)
        x += jnp.dot(gr[...], w0d[...], preferred_element_type=_f32)
        x += jnp.dot(ir[...], w0e[...], preferred_element_type=_f32)
        x = jnp.maximum(x + b0r[...], 0.0)
        x = jnp.maximum(jnp.dot(x, w1r[...], preferred_element_type=_f32)
                        + b1r[...], 0.0)
        x = jnp.maximum(jnp.dot(x, w2r[...], preferred_element_type=_f32)
                        + b2r[...], 0.0)
        x = jnp.dot(x, w3r[...], preferred_element_type=_f32) + b3r[...]
        outr[...] = 1.0 / (1.0 + jnp.exp(-x))

    H2, H4 = 2 * H, 4 * H
    in_specs = [_bspec((RB, H), lambda i: (i, 0))] * 3
    in_specs += [_bspec((RB, VIS), lambda i: (i, 0)),
                 _bspec((RB, 4), lambda i: (i, 0))]
    in_specs += [_bspec(w.shape, lambda i: (0, 0)) for w in w0s]
    in_specs += [_bspec((1, H2), lambda i: (0, 0)),
                 _bspec((H2, H2), lambda i: (0, 0)),
                 _bspec((1, H2), lambda i: (0, 0)),
                 _bspec((H2, H4), lambda i: (0, 0)),
                 _bspec((1, H4), lambda i: (0, 0)),
                 _bspec((H4, 1), lambda i: (0, 0)),
                 _bspec((1, 1), lambda i: (0, 0))]
    return pl.pallas_call(
        body, grid=(GRID_R,),
        in_specs=in_specs,
        out_specs=_bspec((RB, 1), lambda i: (i, 0)),
        out_shape=jax.ShapeDtypeStruct((N, 1), _f32),
        compiler_params=pltpu.CompilerParams(
            dimension_semantics=("arbitrary",)),
    )(o0, o1, o2, goal, info, *w0s,
      v['b0'].reshape(1, H2), v['w1'], v['b1'].reshape(1, H2),
      v['w2'], v['b2'].reshape(1, H4), v['w3'], v['b3'].reshape(1, 1))


def _mlp3(xs, ws0, q):
    y1, s1 = _mm(xs, ws0, q['b0'])
    af1 = _bn_aff(s1, q['g0'], q['be0'])
    y2, s2 = _mm([y1], [q['w1']], q['b1'], affine=af1)
    af2 = _bn_aff(s2, q['g1'], q['be1'])
    return _mm([y2], [q['w2']], q['b2'], affine=af2, relu=False, stats=False)


def kernel(feat, goal_feat, info_feat, adj, params):
    p = params
    fx0 = _mlp3([feat], [p['fe']['w0']], p['fe'])

    def ne(fx, q):
        return _mlp3([fx, goal_feat, info_feat],
                     [q['w0'][:H], q['w0'][H:H + VIS], q['w0'][H + VIS:]], q)

    def a2(a):
        return jnp.stack([a[0, :H], a[0, H:]], axis=1)

    out0 = ne(fx0, p['ne0'])
    h0, st0 = _hst(fx0, p['ga0_W'], a2(p['ga0_a']))
    hp0, rs0 = _gat_dense(adj, st0, h0)
    r0, sr0 = _post_agg(hp0, rs0)
    bs0, bc0 = _bn_aff(sr0, p['bn_g'], p['bn_b'])
    fx1 = _residual(r0, fx0, bs0, bc0)

    out1 = ne(fx1, p['ne1'])
    h1, st1 = _hst(fx1, p['ga1_W'], a2(p['ga1_a']))
    hp1, rs1 = _gat_dense(adj, st1, h1)
    r1, sr1 = _post_agg(hp1, rs1)
    bs1, bc1 = _bn_aff(sr1, p['bn_g'], p['bn_b'])
    fx2 = _residual(r1, fx1, bs1, bc1)

    out2 = ne(fx2, p['ne2'])
    return _vl_head(out0, out1, out2, goal_feat, info_feat, p['vl'])
